# baseline (speedup 1.0000x reference)
import jax
import jax.numpy as jnp
from jax.experimental import pallas as pl
from jax.experimental.pallas import tpu as pltpu

N = 4096
D = 512
DOUT = 512
KN = 16
KS = 16
HID = 128

RB = 256


def _fn_kernel(x_ref, o_ref):
    x = x_ref[...]
    ss = jnp.sum(x * x, axis=1, keepdims=True)
    o_ref[...] = x / (jnp.sqrt(ss) + 1e-12)


def _normalize(feats):
    return pl.pallas_call(
        _fn_kernel,
        grid=(N // 512,),
        in_specs=[pl.BlockSpec((512, D), lambda i: (i, 0))],
        out_specs=pl.BlockSpec((512, D), lambda i: (i, 0)),
        out_shape=jax.ShapeDtypeStruct((N, D), jnp.float32),
    )(feats)


def _simtopk_kernel(lhs_ref, rhs_ref, idx_ref):
    s = jax.lax.dot_general(
        lhs_ref[...], rhs_ref[...],
        dimension_numbers=(((1,), (1,)), ((), ())),
        preferred_element_type=jnp.float32)
    sb = jax.lax.bitcast_convert_type(s, jnp.int32)
    key = jnp.where(sb < 0, sb ^ jnp.int32(0x7FFFFFFF), sb)
    col = jax.lax.broadcasted_iota(jnp.int32, (RB, N), 1)
    outs = []
    for _ in range(KN):
        m = jnp.max(key, axis=1, keepdims=True)
        eq = key == m
        j = jnp.min(jnp.where(eq, col, jnp.int32(N)), axis=1, keepdims=True)
        outs.append(j)
        key = jnp.where(col == j, jnp.int32(-2147483648), key)
    idx_ref[...] = jnp.concatenate(outs, axis=1)


def _sim_topk(fn_bf16):
    return pl.pallas_call(
        _simtopk_kernel,
        grid=(N // RB,),
        in_specs=[
            pl.BlockSpec((RB, D), lambda i: (i, 0)),
            pl.BlockSpec((N, D), lambda i: (0, 0)),
        ],
        out_specs=pl.BlockSpec((RB, KN), lambda i: (i, 0)),
        out_shape=jax.ShapeDtypeStruct((N, KN), jnp.int32),
    )(fn_bf16, fn_bf16)


def _vertex_conv(region, Wkk, bkk, wk1, bk1):
    mult = jnp.einsum('ngd,gjd->ngj', region, Wkk) + bkk
    mult = jax.nn.softmax(mult, axis=-1)
    transformed = jnp.matmul(mult, region)
    pooled = jnp.einsum('nkd,k->nd', transformed, wk1) + bk1
    return pooled


def _tail_kernel(xn_ref, xs_ref, w1_ref, b1_ref, w2_ref, b2_ref,
                 fcw_ref, fcb_ref, out_ref):
    xn = xn_ref[...]
    xs = xs_ref[...]
    hn = jnp.maximum(jnp.dot(xn, w1_ref[...].T,
                             preferred_element_type=jnp.float32) + b1_ref[...], 0.0)
    hs = jnp.maximum(jnp.dot(xs, w1_ref[...].T,
                             preferred_element_type=jnp.float32) + b1_ref[...], 0.0)
    sn = jnp.sum(hn * w2_ref[...], axis=1, keepdims=True) + b2_ref[0, 0]
    ss = jnp.sum(hs * w2_ref[...], axis=1, keepdims=True) + b2_ref[0, 0]
    m = jnp.maximum(sn, ss)
    en = jnp.exp(sn - m)
    es = jnp.exp(ss - m)
    tot = en + es
    x = (en / tot) * xn + (es / tot) * xs
    out_ref[...] = jnp.maximum(
        jnp.dot(x, fcw_ref[...].T, preferred_element_type=jnp.float32)
        + fcb_ref[...], 0.0)


def _tail(xn, xs, ec_w1, ec_b1, ec_w2, ec_b2, fc_w, fc_b):
    return pl.pallas_call(
        _tail_kernel,
        grid=(N // 512,),
        in_specs=[
            pl.BlockSpec((512, D), lambda i: (i, 0)),
            pl.BlockSpec((512, D), lambda i: (i, 0)),
            pl.BlockSpec((HID, D), lambda i: (0, 0)),
            pl.BlockSpec((HID,), lambda i: (0,)),
            pl.BlockSpec((1, HID), lambda i: (0, 0)),
            pl.BlockSpec((1, 1), lambda i: (0, 0)),
            pl.BlockSpec((DOUT, D), lambda i: (0, 0)),
            pl.BlockSpec((DOUT,), lambda i: (0,)),
        ],
        out_specs=pl.BlockSpec((512, DOUT), lambda i: (i, 0)),
        out_shape=jax.ShapeDtypeStruct((N, DOUT), jnp.float32),
    )(xn, xs, ec_w1, ec_b1, ec_w2, ec_b2.reshape(1, 1), fc_w, fc_b)


def kernel(ids, feats, struct_idx, G, ite, fc_w, fc_b,
           vcn_Wkk, vcn_bkk, vcn_wk1, vcn_bk1,
           vcs_Wkk, vcs_bkk, vcs_wk1, vcs_bk1,
           ec_w1, ec_b1, ec_w2, ec_b2):
    fn = feats / (jnp.linalg.norm(feats, axis=1, keepdims=True) + 1e-12)
    nn_idx = _sim_topk(fn.astype(jnp.bfloat16))
    nearest = jnp.take(feats, nn_idx.reshape(-1), axis=0).reshape(N, KN, D)
    xn = _vertex_conv(nearest, vcn_Wkk, vcn_bkk, vcn_wk1, vcn_bk1)
    xn = jnp.where(ite >= 0, xn, jnp.zeros_like(xn))
    region = jnp.take(feats, struct_idx.reshape(-1), axis=0).reshape(N, KS, D)
    xs = _vertex_conv(region, vcs_Wkk, vcs_bkk, vcs_wk1, vcs_bk1)
    xs = jnp.where(ite >= 0, xs, jnp.zeros_like(xs))
    return _tail(xn, xs, ec_w1, ec_b1, ec_w2, ec_b2, fc_w, fc_b)


# speedup vs baseline: 1.5461x; 1.5461x over previous
import functools

import jax
import jax.numpy as jnp
from jax import lax
from jax.experimental import pallas as pl
from jax.experimental.pallas import tpu as pltpu
from jax.experimental.pallas import tpu_sc as plsc

N = 4096
D = 512
DOUT = 512
KN = 16
KS = 16
HID = 128

RB = 256

NC = 2
NS = 16
L = 16
NW = NC * NS
NPT = N // NW
CH = 4
NCH = NPT // CH
ROWS = CH * KN


def _simtopk_kernel(lhs_ref, rhs_ref, idx_ref):
    s = jax.lax.dot_general(
        lhs_ref[...], rhs_ref[...],
        dimension_numbers=(((1,), (1,)), ((), ())),
        preferred_element_type=jnp.float32)
    sb = jax.lax.bitcast_convert_type(s, jnp.int32)
    key = jnp.where(sb < 0, sb ^ jnp.int32(0x7FFFFFFF), sb)
    col = jax.lax.broadcasted_iota(jnp.int32, (RB, N), 1)
    outs = []
    for _ in range(KN):
        m = jnp.max(key, axis=1, keepdims=True)
        eq = key == m
        j = jnp.min(jnp.where(eq, col, jnp.int32(N)), axis=1, keepdims=True)
        outs.append(j)
        key = jnp.where(col == j, jnp.int32(-2147483648), key)
    idx_ref[...] = jnp.concatenate(outs, axis=1)


def _sim_topk(fn_bf16):
    return pl.pallas_call(
        _simtopk_kernel,
        grid=(N // RB,),
        in_specs=[
            pl.BlockSpec((RB, D), lambda i: (i, 0)),
            pl.BlockSpec((N, D), lambda i: (0, 0)),
        ],
        out_specs=pl.BlockSpec((RB, KN), lambda i: (i, 0)),
        out_shape=jax.ShapeDtypeStruct((N, KN), jnp.int32),
    )(fn_bf16, fn_bf16)


def _a2_kernel(f_ref, w_ref, o_ref):
    o_ref[...] = jax.lax.dot_general(
        f_ref[...].astype(jnp.bfloat16), w_ref[...].astype(jnp.bfloat16),
        dimension_numbers=(((1,), (1,)), ((), ())),
        preferred_element_type=jnp.float32)


def _a2(feats, wcat):
    return pl.pallas_call(
        _a2_kernel,
        grid=(N // 512,),
        in_specs=[
            pl.BlockSpec((512, D), lambda i: (i, 0)),
            pl.BlockSpec((2 * KN * KN, D), lambda i: (0, 0)),
        ],
        out_specs=pl.BlockSpec((512, 2 * KN * KN), lambda i: (i, 0)),
        out_shape=jax.ShapeDtypeStruct((N, 2 * KN * KN), jnp.float32),
    )(feats, wcat)


def _sc_conv_body(idxn_hbm, idxs_hbm, a2n_hbm, a2s_hbm, feats_hbm,
                  bkkn_hbm, wkn_hbm, bk1n_hbm,
                  bkks_hbm, wks_hbm, bk1s_hbm,
                  outn_hbm, outs_hbm,
                  idx_v, flat_v, gath_v, c_all, fr_v, pooled_v,
                  bkk_v, wk_v, bk1_v, sem_m, sem_f):
    wid = lax.axis_index("s") * NC + lax.axis_index("c")
    base = wid * NPT
    for (idx_hbm, a2_hbm, bkk_hbm, wk_hbm, bk1_hbm, out_hbm) in (
            (idxn_hbm, a2n_hbm, bkkn_hbm, wkn_hbm, bk1n_hbm, outn_hbm),
            (idxs_hbm, a2s_hbm, bkks_hbm, wks_hbm, bk1s_hbm, outs_hbm)):
        pltpu.sync_copy(bkk_hbm, bkk_v)
        pltpu.sync_copy(wk_hbm, wk_v)
        pltpu.sync_copy(bk1_hbm, bk1_v)
        pltpu.sync_copy(idx_hbm.at[pl.ds(wid * NCH, NCH)], idx_v)
        iot = lax.iota(jnp.int32, L)
        hi8 = jnp.where(iot >= jnp.int32(8), jnp.int32(1), jnp.int32(0))
        for i in range(NPT * KN // L):
            v = idx_v[i // CH, pl.ds((i % CH) * L, L)]
            flat_v[i // 8, pl.ds((i % 8) * L, L)] = v * jnp.int32(2) + hi8

        for sci in range(4):
            handles = []
            for q in range(4):
                handles.append(pltpu.async_copy(
                    a2_hbm.at[flat_v.at[sci * 4 + q]],
                    gath_v.at[pl.ds(q * 128, 128)], sem_m))
            for h in handles:
                h.wait()

            def cfn(i, _, _sci=sci):
                acc = jnp.zeros((L,), jnp.float32)
                for k in range(KN):
                    v = (gath_v[i * KN + k, pl.ds((k % 8) * L, L)]
                         + bkk_v[k, :])
                    m = v
                    for step in (1, 2, 4, 8):
                        m = jnp.maximum(m, m[iot ^ step])
                    e = jnp.exp(v - m)
                    s = e
                    for step in (1, 2, 4, 8):
                        s = s + s[iot ^ step]
                    acc = acc + wk_v[k, :] * (e / s)
                c_all[pl.ds((_sci * 32 + i) * KN, KN)] = acc
                return 0
            lax.fori_loop(0, 32, cfn, 0)

        def bagfn(ch, _):
            pltpu.async_copy(feats_hbm.at[idx_v.at[ch]], fr_v, sem_f).wait()
            for j in range(CH):
                node = ch * CH + j
                cs = c_all[pl.ds(node * KN, KN)]
                ws = [cs[jnp.full((L,), g, jnp.int32)] for g in range(KN)]

                def dfn(dd, u, _j=j, _ws=ws):
                    a = bk1_v[...]
                    for g in range(KN):
                        a = a + _ws[g] * fr_v[_j * KN + g, pl.ds(dd * L, L)]
                    pooled_v[_j, pl.ds(dd * L, L)] = a
                    return 0
                lax.fori_loop(0, D // L, dfn, 0)
            pltpu.sync_copy(pooled_v,
                            out_hbm.at[pl.ds(base + ch * CH, CH)])
            return 0
        lax.fori_loop(0, NCH, bagfn, 0)


def _sc_convs(idxn2d, idxs2d, a2n, a2s, feats,
              bkkn, wkn, bk1n, bkks, wks, bk1s):
    mesh = plsc.VectorSubcoreMesh(core_axis_name="c", subcore_axis_name="s",
                                  num_cores=NC, num_subcores=NS)
    f = pl.kernel(
        _sc_conv_body,
        out_type=[jax.ShapeDtypeStruct((N, D), jnp.float32),
                  jax.ShapeDtypeStruct((N, D), jnp.float32)],
        mesh=mesh,
        scratch_types=[
            pltpu.VMEM((NCH, CH * KN), jnp.int32),
            pltpu.VMEM((16, 128), jnp.int32),
            pltpu.VMEM((512, 128), jnp.float32),
            pltpu.VMEM((NPT * KN,), jnp.float32),
            pltpu.VMEM((ROWS, D), jnp.float32),
            pltpu.VMEM((CH, D), jnp.float32),
            pltpu.VMEM((KN, KN), jnp.float32),
            pltpu.VMEM((KN, KN), jnp.float32),
            pltpu.VMEM((L,), jnp.float32),
            pltpu.SemaphoreType.DMA,
            pltpu.SemaphoreType.DMA,
        ],
    )
    return f(idxn2d, idxs2d, a2n, a2s, feats,
             bkkn, wkn, bk1n, bkks, wks, bk1s)


def _tail_kernel(xn_ref, xs_ref, w1_ref, b1_ref, w2_ref, b2_ref,
                 fcw_ref, fcb_ref, out_ref):
    xn = xn_ref[...]
    xs = xs_ref[...]
    hn = jnp.maximum(jnp.dot(xn, w1_ref[...].T,
                             preferred_element_type=jnp.float32) + b1_ref[...], 0.0)
    hs = jnp.maximum(jnp.dot(xs, w1_ref[...].T,
                             preferred_element_type=jnp.float32) + b1_ref[...], 0.0)
    sn = jnp.sum(hn * w2_ref[...], axis=1, keepdims=True) + b2_ref[0, 0]
    ss = jnp.sum(hs * w2_ref[...], axis=1, keepdims=True) + b2_ref[0, 0]
    m = jnp.maximum(sn, ss)
    en = jnp.exp(sn - m)
    es = jnp.exp(ss - m)
    tot = en + es
    x = (en / tot) * xn + (es / tot) * xs
    out_ref[...] = jnp.maximum(
        jnp.dot(x, fcw_ref[...].T, preferred_element_type=jnp.float32)
        + fcb_ref[...], 0.0)


def _tail(xn, xs, ec_w1, ec_b1, ec_w2, ec_b2, fc_w, fc_b):
    return pl.pallas_call(
        _tail_kernel,
        grid=(N // 512,),
        in_specs=[
            pl.BlockSpec((512, D), lambda i: (i, 0)),
            pl.BlockSpec((512, D), lambda i: (i, 0)),
            pl.BlockSpec((HID, D), lambda i: (0, 0)),
            pl.BlockSpec((HID,), lambda i: (0,)),
            pl.BlockSpec((1, HID), lambda i: (0, 0)),
            pl.BlockSpec((1, 1), lambda i: (0, 0)),
            pl.BlockSpec((DOUT, D), lambda i: (0, 0)),
            pl.BlockSpec((DOUT,), lambda i: (0,)),
        ],
        out_specs=pl.BlockSpec((512, DOUT), lambda i: (i, 0)),
        out_shape=jax.ShapeDtypeStruct((N, DOUT), jnp.float32),
    )(xn, xs, ec_w1, ec_b1, ec_w2, ec_b2.reshape(1, 1), fc_w, fc_b)


def kernel(ids, feats, struct_idx, G, ite, fc_w, fc_b,
           vcn_Wkk, vcn_bkk, vcn_wk1, vcn_bk1,
           vcs_Wkk, vcs_bkk, vcs_wk1, vcs_bk1,
           ec_w1, ec_b1, ec_w2, ec_b2):
    fn = feats / (jnp.linalg.norm(feats, axis=1, keepdims=True) + 1e-12)
    nn_idx = _sim_topk(fn.astype(jnp.bfloat16))

    wcat = jnp.concatenate([vcn_Wkk.reshape(KN * KN, D),
                            vcs_Wkk.reshape(KS * KS, D)], axis=0)
    a2 = _a2(feats, wcat)
    a2n = a2[:, :KN * KN].reshape(N * 2, 128)
    a2s = a2[:, KN * KN:].reshape(N * 2, 128)

    idxn2d = nn_idx.reshape(N // CH, CH * KN)
    idxs2d = struct_idx.reshape(N // CH, CH * KS)
    wknb = jnp.broadcast_to(vcn_wk1[:, None], (KN, KN))
    wksb = jnp.broadcast_to(vcs_wk1[:, None], (KS, KS))
    bk1nv = jnp.broadcast_to(vcn_bk1, (L,))
    bk1sv = jnp.broadcast_to(vcs_bk1, (L,))

    xn, xs = _sc_convs(idxn2d, idxs2d, a2n, a2s, feats,
                       vcn_bkk, wknb, bk1nv, vcs_bkk, wksb, bk1sv)
    xn = jnp.where(ite >= 0, xn, jnp.zeros_like(xn))
    xs = jnp.where(ite >= 0, xs, jnp.zeros_like(xs))
    return _tail(xn, xs, ec_w1, ec_b1, ec_w2, ec_b2, fc_w, fc_b)


# speedup vs baseline: 1.7874x; 1.1561x over previous
import functools

import jax
import jax.numpy as jnp
from jax import lax
from jax.experimental import pallas as pl
from jax.experimental.pallas import tpu as pltpu
from jax.experimental.pallas import tpu_sc as plsc

N = 4096
D = 512
DOUT = 512
KN = 16
KS = 16
HID = 128

RB = 256

NC = 2
NS = 16
L = 16
NW = NC * NS
NPT = N // NW
CH = 2
NCH = NPT // CH
ROWS = CH * KN


def _simtopk_kernel(lhs_ref, rhs_ref, idx_ref):
    s = jax.lax.dot_general(
        lhs_ref[...], rhs_ref[...],
        dimension_numbers=(((1,), (1,)), ((), ())),
        preferred_element_type=jnp.float32)
    sb = jax.lax.bitcast_convert_type(s, jnp.int32)
    key = jnp.where(sb < 0, sb ^ jnp.int32(0x7FFFFFFF), sb)
    col = jax.lax.broadcasted_iota(jnp.int32, (RB, N), 1)
    outs = []
    for _ in range(KN):
        m = jnp.max(key, axis=1, keepdims=True)
        eq = key == m
        j = jnp.min(jnp.where(eq, col, jnp.int32(N)), axis=1, keepdims=True)
        outs.append(j)
        key = jnp.where(col == j, jnp.int32(-2147483648), key)
    idx_ref[...] = jnp.concatenate(outs, axis=1)


def _sim_topk(fn_bf16):
    return pl.pallas_call(
        _simtopk_kernel,
        grid=(N // RB,),
        in_specs=[
            pl.BlockSpec((RB, D), lambda i: (i, 0)),
            pl.BlockSpec((N, D), lambda i: (0, 0)),
        ],
        out_specs=pl.BlockSpec((RB, KN), lambda i: (i, 0)),
        out_shape=jax.ShapeDtypeStruct((N, KN), jnp.int32),
    )(fn_bf16, fn_bf16)


def _a2_kernel(f_ref, w_ref, o_ref):
    o_ref[...] = jax.lax.dot_general(
        f_ref[...].astype(jnp.bfloat16), w_ref[...].astype(jnp.bfloat16),
        dimension_numbers=(((1,), (1,)), ((), ())),
        preferred_element_type=jnp.float32)


def _a2(feats, wcat):
    return pl.pallas_call(
        _a2_kernel,
        grid=(N // 512,),
        in_specs=[
            pl.BlockSpec((512, D), lambda i: (i, 0)),
            pl.BlockSpec((2 * KN * KN, D), lambda i: (0, 0)),
        ],
        out_specs=pl.BlockSpec((512, 2 * KN * KN), lambda i: (i, 0)),
        out_shape=jax.ShapeDtypeStruct((N, 2 * KN * KN), jnp.float32),
    )(feats, wcat)


SCN = 16
NPS = NPT // SCN


def _sc_conv_body(idx2d_hbm, a2_hbm, feats_hbm, bkk_hbm, wk_hbm, bk1_hbm,
                  out_hbm,
                  idx_v, flat_v, gath_a, gath_b, c_all, fr_a, fr_b,
                  pooled_v, bkk_v, wk_v, bk1_v,
                  sem_ga, sem_gb, sem_fa, sem_fb):
    wid = lax.axis_index("s") * NC + lax.axis_index("c")
    base = wid * NPT
    pltpu.sync_copy(bkk_hbm, bkk_v)
    pltpu.sync_copy(wk_hbm, wk_v)
    pltpu.sync_copy(bk1_hbm, bk1_v)
    pltpu.sync_copy(idx2d_hbm.at[pl.ds(wid * NCH, NCH)], idx_v)
    iot = lax.iota(jnp.int32, L)
    hi8 = jnp.where(iot >= jnp.int32(8), jnp.int32(1), jnp.int32(0))
    for i in range(NPT * KN // L):
        v = idx_v[i // CH, pl.ds((i % CH) * L, L)]
        flat_v[i // 8, pl.ds((i % 8) * L, L)] = v * jnp.int32(2) + hi8

    gbufs = (gath_a, gath_b)
    gsems = (sem_ga, sem_gb)

    def _fire_gath(sci):
        buf = gbufs[sci % 2]
        sem = gsems[sci % 2]
        return [pltpu.async_copy(a2_hbm.at[flat_v.at[sci]], buf, sem)]

    pend = {0: _fire_gath(0)}
    for sci in range(SCN):
        if sci + 1 < SCN:
            pend[sci + 1] = _fire_gath(sci + 1)
        for h in pend.pop(sci):
            h.wait()
        buf = gbufs[sci % 2]

        def cfn(i, _, _sci=sci, _buf=buf):
            acc = jnp.zeros((L,), jnp.float32)
            for k in range(KN):
                v = _buf[i * KN + k, pl.ds((k % 8) * L, L)] + bkk_v[k, :]
                m = v
                for step in (1, 2, 4, 8):
                    m = jnp.maximum(m, m[iot ^ step])
                e = jnp.exp(v - m)
                s = e
                for step in (1, 2, 4, 8):
                    s = s + s[iot ^ step]
                acc = acc + wk_v[k, :] * (e / s)
            c_all[pl.ds((_sci * NPS + i) * KN, KN)] = acc
            return 0
        lax.fori_loop(0, NPS, cfn, 0)

    def _bag_compute(ch, fr_ref):
        for j in range(CH):
            node = ch * CH + j
            cs = c_all[pl.ds(node * KN, KN)]
            ws = [cs[jnp.full((L,), g, jnp.int32)] for g in range(KN)]

            def dfn(dd, u, _j=j, _ws=ws, _fr=fr_ref):
                a = bk1_v[...]
                for g in range(KN):
                    a = a + _ws[g] * _fr[_j * KN + g, pl.ds(dd * L, L)]
                pooled_v[_j, pl.ds(dd * L, L)] = a
                return 0
            lax.fori_loop(0, D // L, dfn, 0)
        pltpu.sync_copy(pooled_v, out_hbm.at[pl.ds(base + ch * CH, CH)])

    pltpu.async_copy(feats_hbm.at[idx_v.at[0]], fr_a, sem_fa)

    def bag2(t, _):
        ch0 = t * 2
        pltpu.async_copy(feats_hbm.at[idx_v.at[ch0 + 1]], fr_b, sem_fb)
        pltpu.make_async_copy(feats_hbm.at[idx_v.at[ch0]], fr_a, sem_fa).wait()
        _bag_compute(ch0, fr_a)

        @pl.when(ch0 + 2 < NCH)
        def _():
            pltpu.async_copy(feats_hbm.at[idx_v.at[ch0 + 2]], fr_a, sem_fa)
        pltpu.make_async_copy(feats_hbm.at[idx_v.at[ch0 + 1]], fr_b,
                              sem_fb).wait()
        _bag_compute(ch0 + 1, fr_b)
        return 0
    lax.fori_loop(0, NCH // 2, bag2, 0)


def _sc_conv(idx2d, a2t, feats, bkk, wkb, bk1v):
    mesh = plsc.VectorSubcoreMesh(core_axis_name="c", subcore_axis_name="s",
                                  num_cores=NC, num_subcores=NS)
    f = pl.kernel(
        _sc_conv_body,
        out_type=jax.ShapeDtypeStruct((N, D), jnp.float32),
        mesh=mesh,
        scratch_types=[
            pltpu.VMEM((NCH, CH * KN), jnp.int32),
            pltpu.VMEM((16, 128), jnp.int32),
            pltpu.VMEM((NPS * KN, 128), jnp.float32),
            pltpu.VMEM((NPS * KN, 128), jnp.float32),
            pltpu.VMEM((NPT * KN,), jnp.float32),
            pltpu.VMEM((ROWS, D), jnp.float32),
            pltpu.VMEM((ROWS, D), jnp.float32),
            pltpu.VMEM((CH, D), jnp.float32),
            pltpu.VMEM((KN, KN), jnp.float32),
            pltpu.VMEM((KN, KN), jnp.float32),
            pltpu.VMEM((L,), jnp.float32),
            pltpu.SemaphoreType.DMA,
            pltpu.SemaphoreType.DMA,
            pltpu.SemaphoreType.DMA,
            pltpu.SemaphoreType.DMA,
        ],
    )
    return f(idx2d, a2t, feats, bkk, wkb, bk1v)


def _tail_kernel(xn_ref, xs_ref, w1_ref, b1_ref, w2_ref, b2_ref,
                 fcw_ref, fcb_ref, out_ref):
    xn = xn_ref[...]
    xs = xs_ref[...]
    hn = jnp.maximum(jnp.dot(xn, w1_ref[...].T,
                             preferred_element_type=jnp.float32) + b1_ref[...], 0.0)
    hs = jnp.maximum(jnp.dot(xs, w1_ref[...].T,
                             preferred_element_type=jnp.float32) + b1_ref[...], 0.0)
    sn = jnp.sum(hn * w2_ref[...], axis=1, keepdims=True) + b2_ref[0, 0]
    ss = jnp.sum(hs * w2_ref[...], axis=1, keepdims=True) + b2_ref[0, 0]
    m = jnp.maximum(sn, ss)
    en = jnp.exp(sn - m)
    es = jnp.exp(ss - m)
    tot = en + es
    x = (en / tot) * xn + (es / tot) * xs
    out_ref[...] = jnp.maximum(
        jnp.dot(x, fcw_ref[...].T, preferred_element_type=jnp.float32)
        + fcb_ref[...], 0.0)


def _tail(xn, xs, ec_w1, ec_b1, ec_w2, ec_b2, fc_w, fc_b):
    return pl.pallas_call(
        _tail_kernel,
        grid=(N // 512,),
        in_specs=[
            pl.BlockSpec((512, D), lambda i: (i, 0)),
            pl.BlockSpec((512, D), lambda i: (i, 0)),
            pl.BlockSpec((HID, D), lambda i: (0, 0)),
            pl.BlockSpec((HID,), lambda i: (0,)),
            pl.BlockSpec((1, HID), lambda i: (0, 0)),
            pl.BlockSpec((1, 1), lambda i: (0, 0)),
            pl.BlockSpec((DOUT, D), lambda i: (0, 0)),
            pl.BlockSpec((DOUT,), lambda i: (0,)),
        ],
        out_specs=pl.BlockSpec((512, DOUT), lambda i: (i, 0)),
        out_shape=jax.ShapeDtypeStruct((N, DOUT), jnp.float32),
    )(xn, xs, ec_w1, ec_b1, ec_w2, ec_b2.reshape(1, 1), fc_w, fc_b)


def kernel(ids, feats, struct_idx, G, ite, fc_w, fc_b,
           vcn_Wkk, vcn_bkk, vcn_wk1, vcn_bk1,
           vcs_Wkk, vcs_bkk, vcs_wk1, vcs_bk1,
           ec_w1, ec_b1, ec_w2, ec_b2):
    fn = feats / (jnp.linalg.norm(feats, axis=1, keepdims=True) + 1e-12)
    nn_idx = _sim_topk(fn.astype(jnp.bfloat16))

    wcat = jnp.concatenate([vcn_Wkk.reshape(KN * KN, D),
                            vcs_Wkk.reshape(KS * KS, D)], axis=0)
    a2 = _a2(feats, wcat)
    a2n = a2[:, :KN * KN].reshape(N * 2, 128)
    a2s = a2[:, KN * KN:].reshape(N * 2, 128)

    idxn2d = nn_idx.reshape(N // CH, CH * KN)
    idxs2d = struct_idx.reshape(N // CH, CH * KS)
    wknb = jnp.broadcast_to(vcn_wk1[:, None], (KN, KN))
    wksb = jnp.broadcast_to(vcs_wk1[:, None], (KS, KS))
    bk1nv = jnp.broadcast_to(vcn_bk1, (L,))
    bk1sv = jnp.broadcast_to(vcs_bk1, (L,))

    xs = _sc_conv(idxs2d, a2s, feats, vcs_bkk, wksb, bk1sv)
    xn = _sc_conv(idxn2d, a2n, feats, vcn_bkk, wknb, bk1nv)
    xn = jnp.where(ite >= 0, xn, jnp.zeros_like(xn))
    xs = jnp.where(ite >= 0, xs, jnp.zeros_like(xs))
    return _tail(xn, xs, ec_w1, ec_b1, ec_w2, ec_b2, fc_w, fc_b)


# speedup vs baseline: 1.8140x; 1.0149x over previous
import functools

import jax
import jax.numpy as jnp
from jax import lax
from jax.experimental import pallas as pl
from jax.experimental.pallas import tpu as pltpu
from jax.experimental.pallas import tpu_sc as plsc

N = 4096
D = 512
DOUT = 512
KN = 16
KS = 16
HID = 128

RB = 256

NC = 2
NS = 16
L = 16
NW = NC * NS
NPT = N // NW
CH = 2
NCH = NPT // CH
ROWS = CH * KN


def _simtopk_kernel(lhs_ref, rhs_ref, idx_ref):
    s = jax.lax.dot_general(
        lhs_ref[...], rhs_ref[...],
        dimension_numbers=(((1,), (1,)), ((), ())),
        preferred_element_type=jnp.float32)
    sb = jax.lax.bitcast_convert_type(s, jnp.int32)
    key = jnp.where(sb < 0, sb ^ jnp.int32(0x7FFFFFFF), sb)
    col = jax.lax.broadcasted_iota(jnp.int32, (RB, N), 1)
    outs = []
    for _ in range(KN):
        m = jnp.max(key, axis=1, keepdims=True)
        eq = key == m
        j = jnp.min(jnp.where(eq, col, jnp.int32(N)), axis=1, keepdims=True)
        outs.append(j)
        key = jnp.where(col == j, jnp.int32(-2147483648), key)
    idx_ref[...] = jnp.concatenate(outs, axis=1)


def _sim_topk(fn_bf16):
    return pl.pallas_call(
        _simtopk_kernel,
        grid=(N // RB,),
        in_specs=[
            pl.BlockSpec((RB, D), lambda i: (i, 0)),
            pl.BlockSpec((N, D), lambda i: (0, 0)),
        ],
        out_specs=pl.BlockSpec((RB, KN), lambda i: (i, 0)),
        out_shape=jax.ShapeDtypeStruct((N, KN), jnp.int32),
    )(fn_bf16, fn_bf16)


def _a2_kernel(f_ref, w_ref, o_ref):
    o_ref[...] = jax.lax.dot_general(
        f_ref[...].astype(jnp.bfloat16), w_ref[...].astype(jnp.bfloat16),
        dimension_numbers=(((1,), (1,)), ((), ())),
        preferred_element_type=jnp.float32)


def _a2(feats, wcat):
    return pl.pallas_call(
        _a2_kernel,
        grid=(N // 512,),
        in_specs=[
            pl.BlockSpec((512, D), lambda i: (i, 0)),
            pl.BlockSpec((2 * KN * KN, D), lambda i: (0, 0)),
        ],
        out_specs=pl.BlockSpec((512, 2 * KN * KN), lambda i: (i, 0)),
        out_shape=jax.ShapeDtypeStruct((N, 2 * KN * KN), jnp.float32),
    )(feats, wcat)


SCN = 16
NPS = NPT // SCN


def _sc_conv_body(idx2d_hbm, a2_hbm, feats_hbm, bkk_hbm, wk_hbm, bk1_hbm,
                  out_hbm,
                  idx_v, flat_v, gath_a, gath_b, c_all, fr_a, fr_b,
                  pooled_v, bkk_v, wk_v, bk1_v,
                  sem_ga, sem_gb, sem_fa, sem_fb):
    wid = lax.axis_index("s") * NC + lax.axis_index("c")
    base = wid * NPT
    pltpu.sync_copy(bkk_hbm, bkk_v)
    pltpu.sync_copy(wk_hbm, wk_v)
    pltpu.sync_copy(bk1_hbm, bk1_v)
    pltpu.sync_copy(idx2d_hbm.at[pl.ds(wid * NCH, NCH)], idx_v)
    iot = lax.iota(jnp.int32, L)
    hi8 = jnp.where(iot >= jnp.int32(8), jnp.int32(1), jnp.int32(0))
    for i in range(NPT * KN // L):
        v = idx_v[i // CH, pl.ds((i % CH) * L, L)]
        flat_v[i // 8, pl.ds((i % 8) * L, L)] = v * jnp.int32(2) + hi8

    gbufs = (gath_a, gath_b)
    gsems = (sem_ga, sem_gb)

    def _fire_gath(sci):
        buf = gbufs[sci % 2]
        sem = gsems[sci % 2]
        return [pltpu.async_copy(a2_hbm.at[flat_v.at[sci]], buf, sem)]

    pend = {0: _fire_gath(0)}
    for sci in range(SCN):
        if sci + 1 < SCN:
            pend[sci + 1] = _fire_gath(sci + 1)
        for h in pend.pop(sci):
            h.wait()
        buf = gbufs[sci % 2]

        def cfn(i, _, _sci=sci, _buf=buf):
            acc = jnp.zeros((L,), jnp.float32)
            for k in range(KN):
                v = _buf[i * KN + k, pl.ds((k % 8) * L, L)] + bkk_v[k, :]
                m = v
                for step in (1, 2, 4, 8):
                    m = jnp.maximum(m, m[iot ^ step])
                e = jnp.exp(v - m)
                s = e
                for step in (1, 2, 4, 8):
                    s = s + s[iot ^ step]
                acc = acc + wk_v[k, :] * (e / s)
            c_all[pl.ds((_sci * NPS + i) * KN, KN)] = acc
            return 0
        lax.fori_loop(0, NPS, cfn, 0)

    def _bag_compute(ch, fr_ref):
        for j in range(CH):
            node = ch * CH + j
            cs = c_all[pl.ds(node * KN, KN)]
            ws = [cs[jnp.full((L,), g, jnp.int32)] for g in range(KN)]

            def dfn(dd, u, _j=j, _ws=ws, _fr=fr_ref):
                a = bk1_v[...]
                for g in range(KN):
                    a = a + _ws[g] * _fr[_j * KN + g, pl.ds(dd * L, L)]
                pooled_v[_j, pl.ds(dd * L, L)] = a
                return 0
            lax.fori_loop(0, D // L, dfn, 0)
        pltpu.sync_copy(pooled_v, out_hbm.at[pl.ds(base + ch * CH, CH)])

    pltpu.async_copy(feats_hbm.at[idx_v.at[0]], fr_a, sem_fa)

    def bag2(t, _):
        ch0 = t * 2
        pltpu.async_copy(feats_hbm.at[idx_v.at[ch0 + 1]], fr_b, sem_fb)
        pltpu.make_async_copy(feats_hbm.at[idx_v.at[ch0]], fr_a, sem_fa).wait()
        _bag_compute(ch0, fr_a)

        @pl.when(ch0 + 2 < NCH)
        def _():
            pltpu.async_copy(feats_hbm.at[idx_v.at[ch0 + 2]], fr_a, sem_fa)
        pltpu.make_async_copy(feats_hbm.at[idx_v.at[ch0 + 1]], fr_b,
                              sem_fb).wait()
        _bag_compute(ch0 + 1, fr_b)
        return 0
    lax.fori_loop(0, NCH // 2, bag2, 0)


def _sc_conv(idx2d, a2t, feats, bkk, wkb, bk1v):
    mesh = plsc.VectorSubcoreMesh(core_axis_name="c", subcore_axis_name="s",
                                  num_cores=NC, num_subcores=NS)
    f = pl.kernel(
        _sc_conv_body,
        out_type=jax.ShapeDtypeStruct((N, D), jnp.float32),
        mesh=mesh,
        scratch_types=[
            pltpu.VMEM((NCH, CH * KN), jnp.int32),
            pltpu.VMEM((16, 128), jnp.int32),
            pltpu.VMEM((NPS * KN, 128), jnp.float32),
            pltpu.VMEM((NPS * KN, 128), jnp.float32),
            pltpu.VMEM((NPT * KN,), jnp.float32),
            pltpu.VMEM((ROWS, D), jnp.float32),
            pltpu.VMEM((ROWS, D), jnp.float32),
            pltpu.VMEM((CH, D), jnp.float32),
            pltpu.VMEM((KN, KN), jnp.float32),
            pltpu.VMEM((KN, KN), jnp.float32),
            pltpu.VMEM((L,), jnp.float32),
            pltpu.SemaphoreType.DMA,
            pltpu.SemaphoreType.DMA,
            pltpu.SemaphoreType.DMA,
            pltpu.SemaphoreType.DMA,
        ],
    )
    return f(idx2d, a2t, feats, bkk, wkb, bk1v)


def _tail_kernel(xn_ref, xs_ref, g_ref, w1_ref, b1_ref, w2_ref, b2_ref,
                 fcw_ref, fcb_ref, out_ref):
    xn = xn_ref[...] * g_ref[0, 0]
    xs = xs_ref[...] * g_ref[0, 0]
    hn = jnp.maximum(jnp.dot(xn, w1_ref[...].T,
                             preferred_element_type=jnp.float32) + b1_ref[...], 0.0)
    hs = jnp.maximum(jnp.dot(xs, w1_ref[...].T,
                             preferred_element_type=jnp.float32) + b1_ref[...], 0.0)
    sn = jnp.sum(hn * w2_ref[...], axis=1, keepdims=True) + b2_ref[0, 0]
    ss = jnp.sum(hs * w2_ref[...], axis=1, keepdims=True) + b2_ref[0, 0]
    m = jnp.maximum(sn, ss)
    en = jnp.exp(sn - m)
    es = jnp.exp(ss - m)
    tot = en + es
    x = (en / tot) * xn + (es / tot) * xs
    out_ref[...] = jnp.maximum(
        jnp.dot(x, fcw_ref[...].T, preferred_element_type=jnp.float32)
        + fcb_ref[...], 0.0)


def _tail(xn, xs, gate, ec_w1, ec_b1, ec_w2, ec_b2, fc_w, fc_b):
    return pl.pallas_call(
        _tail_kernel,
        grid=(N // 512,),
        in_specs=[
            pl.BlockSpec((512, D), lambda i: (i, 0)),
            pl.BlockSpec((512, D), lambda i: (i, 0)),
            pl.BlockSpec((1, 1), lambda i: (0, 0)),
            pl.BlockSpec((HID, D), lambda i: (0, 0)),
            pl.BlockSpec((HID,), lambda i: (0,)),
            pl.BlockSpec((1, HID), lambda i: (0, 0)),
            pl.BlockSpec((1, 1), lambda i: (0, 0)),
            pl.BlockSpec((DOUT, D), lambda i: (0, 0)),
            pl.BlockSpec((DOUT,), lambda i: (0,)),
        ],
        out_specs=pl.BlockSpec((512, DOUT), lambda i: (i, 0)),
        out_shape=jax.ShapeDtypeStruct((N, DOUT), jnp.float32),
    )(xn, xs, gate, ec_w1, ec_b1, ec_w2, ec_b2.reshape(1, 1), fc_w, fc_b)


def kernel(ids, feats, struct_idx, G, ite, fc_w, fc_b,
           vcn_Wkk, vcn_bkk, vcn_wk1, vcn_bk1,
           vcs_Wkk, vcs_bkk, vcs_wk1, vcs_bk1,
           ec_w1, ec_b1, ec_w2, ec_b2):
    wcat = jnp.concatenate([vcn_Wkk.reshape(KN * KN, D),
                            vcs_Wkk.reshape(KS * KS, D)], axis=0)
    a2 = _a2(feats, wcat)
    a2n = a2[:, :KN * KN].reshape(N * 2, 128)
    a2s = a2[:, KN * KN:].reshape(N * 2, 128)

    idxs2d = struct_idx.reshape(N // CH, CH * KS)
    wksb = jnp.broadcast_to(vcs_wk1[:, None], (KS, KS))
    bk1sv = jnp.broadcast_to(vcs_bk1, (L,))
    xs = _sc_conv(idxs2d, a2s, feats, vcs_bkk, wksb, bk1sv)

    fn = feats / (jnp.linalg.norm(feats, axis=1, keepdims=True) + 1e-12)
    nn_idx = _sim_topk(fn.astype(jnp.bfloat16))

    idxn2d = nn_idx.reshape(N // CH, CH * KN)
    wknb = jnp.broadcast_to(vcn_wk1[:, None], (KN, KN))
    bk1nv = jnp.broadcast_to(vcn_bk1, (L,))
    xn = _sc_conv(idxn2d, a2n, feats, vcn_bkk, wknb, bk1nv)

    gate = (ite >= 0).astype(jnp.float32).reshape(1, 1)
    return _tail(xn, xs, gate, ec_w1, ec_b1, ec_w2, ec_b2, fc_w, fc_b)
